# baseline (speedup 1.0000x reference)
import functools

import jax
import jax.numpy as jnp
from jax import lax
from jax.experimental import pallas as pl
from jax.experimental.pallas import tpu as pltpu
from jax.experimental.pallas import tpu_sc as plsc

VOCAB = 100000
EMBED_DIM = 64
BATCH = 1024
CTX_LEN = 20

LANES = 16
IDX_CHUNK = 128


def _pooled_sc(context, emb_table):
    info = plsc.get_sparse_core_info()
    nw = info.num_cores * info.num_subcores
    b_per_w = BATCH // nw
    d_groups = EMBED_DIM // LANES

    mesh = plsc.VectorSubcoreMesh(core_axis_name="c", subcore_axis_name="s")

    @functools.partial(
        pl.kernel,
        out_type=jax.ShapeDtypeStruct((BATCH, EMBED_DIM), jnp.float32),
        mesh=mesh,
        scratch_types=[
            pltpu.VMEM((b_per_w, CTX_LEN), jnp.int32),
            pltpu.VMEM((b_per_w, CTX_LEN, 128), jnp.float32),
            pltpu.VMEM((b_per_w, EMBED_DIM), jnp.float32),
            pltpu.SemaphoreType.DMA,
        ],
        compiler_params=pltpu.CompilerParams(use_tc_tiling_on_sc=True),
    )
    def sc_pool(ctx_hbm, table_hbm, out_hbm, idx_v, rows_v, out_v, sem):
        wid = lax.axis_index("s") * info.num_cores + lax.axis_index("c")
        base = wid * b_per_w
        pltpu.sync_copy(ctx_hbm.at[pl.ds(base, b_per_w)], idx_v)
        copies = []
        for i in range(b_per_w):
            copies.append(
                pltpu.make_async_copy(
                    table_hbm.at[idx_v.at[i]],
                    rows_v.at[i],
                    sem,
                )
            )
            copies[-1].start()
        for c in copies:
            c.wait()

        scale = jnp.float32(1.0 / CTX_LEN)

        def body(b, carry):
            for d in range(d_groups):
                sl = pl.ds(d * LANES, LANES)
                acc = rows_v[b, 0, sl]
                for l in range(1, CTX_LEN):
                    acc = acc + rows_v[b, l, sl]
                out_v[b, sl] = acc * scale
            return carry

        lax.fori_loop(0, b_per_w, body, 0)
        pltpu.sync_copy(out_v, out_hbm.at[pl.ds(base, b_per_w)])

    return sc_pool(context >> 1, emb_table.reshape(VOCAB // 2, 128))


BV = 5120
NB = pl.cdiv(VOCAB, BV)


def _proj_body(pooled_ref, w_ref, b_ref, out_ref):
    acc = lax.dot_general(
        w_ref[...],
        pooled_ref[...],
        (((1,), (1,)), ((), ())),
        preferred_element_type=jnp.float32,
    )
    out_ref[...] = acc + b_ref[...]


def _proj_tc(pooled, W, b):
    return pl.pallas_call(
        _proj_body,
        grid=(NB,),
        in_specs=[
            pl.BlockSpec((BATCH, EMBED_DIM), lambda j: (0, 0)),
            pl.BlockSpec((BV, EMBED_DIM), lambda j: (j, 0)),
            pl.BlockSpec((BV, 1), lambda j: (j, 0)),
        ],
        out_specs=pl.BlockSpec((BV, BATCH), lambda j: (j, 0)),
        out_shape=jax.ShapeDtypeStruct((VOCAB, BATCH), jnp.float32),
        compiler_params=pltpu.CompilerParams(
            dimension_semantics=("arbitrary",),
        ),
    )(pooled, W, b.reshape(VOCAB, 1))


def kernel(context, emb_table, W, b):
    pooled = _pooled_sc(context, emb_table)
    return _proj_tc(pooled, W, b).T


# speedup vs baseline: 1.4355x; 1.4355x over previous
import functools

import jax
import jax.numpy as jnp
from jax import lax
from jax.experimental import pallas as pl
from jax.experimental.pallas import tpu as pltpu
from jax.experimental.pallas import tpu_sc as plsc

VOCAB = 100000
EMBED_DIM = 64
BATCH = 1024
CTX_LEN = 20

LANES = 16
POOL_W = 128


def _pooled_sc(context, emb_table):
    info = plsc.get_sparse_core_info()
    nw = info.num_cores * info.num_subcores
    b_per_w = BATCH // nw
    d_groups = EMBED_DIM // LANES

    mesh = plsc.VectorSubcoreMesh(core_axis_name="c", subcore_axis_name="s")

    @functools.partial(
        pl.kernel,
        out_type=jax.ShapeDtypeStruct((BATCH, POOL_W), jnp.float32),
        mesh=mesh,
        scratch_types=[
            pltpu.VMEM((b_per_w, CTX_LEN), jnp.int32),
            pltpu.VMEM((b_per_w, CTX_LEN, EMBED_DIM), jnp.float32),
            pltpu.VMEM((b_per_w, POOL_W), jnp.float32),
            pltpu.SemaphoreType.DMA,
        ],
        compiler_params=pltpu.CompilerParams(use_tc_tiling_on_sc=False),
    )
    def sc_pool(ctx_hbm, table_hbm, out_hbm, idx_v, rows_v, out_v, sem):
        wid = lax.axis_index("s") * info.num_cores + lax.axis_index("c")
        base = wid * b_per_w
        pltpu.sync_copy(ctx_hbm.at[pl.ds(base, b_per_w)], idx_v)
        copies = []
        for i in range(b_per_w):
            copies.append(
                pltpu.make_async_copy(
                    table_hbm.at[idx_v.at[i]],
                    rows_v.at[i],
                    sem,
                )
            )
            copies[-1].start()
        for c in copies:
            c.wait()

        scale = jnp.float32(1.0 / CTX_LEN)
        ones_col = jnp.where(
            lax.iota(jnp.int32, LANES) == 0, jnp.float32(1.0), jnp.float32(0.0)
        )

        def body(b, carry):
            for d in range(d_groups):
                sl = pl.ds(d * LANES, LANES)
                acc = rows_v[b, 0, sl]
                for l in range(1, CTX_LEN):
                    acc = acc + rows_v[b, l, sl]
                out_v[b, sl] = acc * scale
            out_v[b, pl.ds(EMBED_DIM, LANES)] = ones_col
            out_v[b, pl.ds(EMBED_DIM + LANES, LANES)] = ones_col * 0.0
            out_v[b, pl.ds(EMBED_DIM + 2 * LANES, LANES)] = ones_col * 0.0
            out_v[b, pl.ds(EMBED_DIM + 3 * LANES, LANES)] = ones_col * 0.0
            return carry

        lax.fori_loop(0, b_per_w, body, 0)
        pltpu.sync_copy(out_v, out_hbm.at[pl.ds(base, b_per_w)])

    return sc_pool(context, emb_table)


BV = 4096
NB = pl.cdiv(VOCAB, BV)


def _proj_body(pooled_ref, wt_ref, b_ref, out_ref):
    w_aug = jnp.concatenate(
        [
            wt_ref[...],
            b_ref[...],
            jnp.zeros((POOL_W - EMBED_DIM - 1, BV), jnp.float32),
        ],
        axis=0,
    )
    out_ref[...] = lax.dot_general(
        w_aug,
        pooled_ref[...],
        (((0,), (1,)), ((), ())),
        preferred_element_type=jnp.float32,
    )


def _proj_tc(pooled_aug, Wt, b2):
    return pl.pallas_call(
        _proj_body,
        grid=(NB,),
        in_specs=[
            pl.BlockSpec((BATCH, POOL_W), lambda j: (0, 0)),
            pl.BlockSpec((EMBED_DIM, BV), lambda j: (0, j)),
            pl.BlockSpec((1, BV), lambda j: (0, j)),
        ],
        out_specs=pl.BlockSpec((BV, BATCH), lambda j: (j, 0)),
        out_shape=jax.ShapeDtypeStruct((VOCAB, BATCH), jnp.float32),
        compiler_params=pltpu.CompilerParams(
            dimension_semantics=("arbitrary",),
        ),
    )(pooled_aug, Wt, b2)


def kernel(context, emb_table, W, b):
    pooled_aug = _pooled_sc(context, emb_table)
    return _proj_tc(pooled_aug, W.T, b.reshape(1, VOCAB)).T


# speedup vs baseline: 1.4391x; 1.0025x over previous
import functools

import jax
import jax.numpy as jnp
from jax import lax
from jax.experimental import pallas as pl
from jax.experimental.pallas import tpu as pltpu
from jax.experimental.pallas import tpu_sc as plsc

VOCAB = 100000
EMBED_DIM = 64
BATCH = 1024
CTX_LEN = 20

LANES = 16
POOL_W = 128


def _pooled_sc(context, emb_table):
    info = plsc.get_sparse_core_info()
    nw = info.num_cores * info.num_subcores
    b_per_w = BATCH // nw
    d_groups = EMBED_DIM // LANES

    mesh = plsc.VectorSubcoreMesh(core_axis_name="c", subcore_axis_name="s")

    idx_per_w = b_per_w * CTX_LEN
    n_chunks = idx_per_w // 128
    ctx3 = context.reshape(nw, n_chunks, 128)

    @functools.partial(
        pl.kernel,
        out_type=jax.ShapeDtypeStruct((BATCH, POOL_W), jnp.float32),
        mesh=mesh,
        scratch_types=[
            pltpu.VMEM((n_chunks, 128), jnp.int32),
            pltpu.VMEM((idx_per_w, EMBED_DIM), jnp.float32),
            pltpu.VMEM((b_per_w, POOL_W), jnp.float32),
            pltpu.SemaphoreType.DMA,
        ],
        compiler_params=pltpu.CompilerParams(use_tc_tiling_on_sc=False),
    )
    def sc_pool(ctx_hbm, table_hbm, out_hbm, idx_v, rows_v, out_v, sem):
        wid = lax.axis_index("s") * info.num_cores + lax.axis_index("c")
        base = wid * b_per_w
        pltpu.sync_copy(ctx_hbm.at[wid], idx_v)
        copies = []
        for j in range(n_chunks):
            copies.append(
                pltpu.make_async_copy(
                    table_hbm.at[idx_v.at[j]],
                    rows_v.at[pl.ds(j * 128, 128)],
                    sem,
                )
            )
            copies[-1].start()
        for c in copies:
            c.wait()

        scale = jnp.float32(1.0 / CTX_LEN)
        ones_col = jnp.where(
            lax.iota(jnp.int32, LANES) == 0, jnp.float32(1.0), jnp.float32(0.0)
        )

        def body(b, carry):
            rbase = b * CTX_LEN
            for d in range(d_groups):
                sl = pl.ds(d * LANES, LANES)
                acc = rows_v[rbase, sl]
                for l in range(1, CTX_LEN):
                    acc = acc + rows_v[rbase + l, sl]
                out_v[b, sl] = acc * scale
            out_v[b, pl.ds(EMBED_DIM, LANES)] = ones_col
            out_v[b, pl.ds(EMBED_DIM + LANES, LANES)] = ones_col * 0.0
            out_v[b, pl.ds(EMBED_DIM + 2 * LANES, LANES)] = ones_col * 0.0
            out_v[b, pl.ds(EMBED_DIM + 3 * LANES, LANES)] = ones_col * 0.0
            return carry

        lax.fori_loop(0, b_per_w, body, 0)
        pltpu.sync_copy(out_v, out_hbm.at[pl.ds(base, b_per_w)])

    return sc_pool(ctx3, emb_table)


BV = 4096
NB = pl.cdiv(VOCAB, BV)


def _proj_body(pooled_ref, wt_ref, b_ref, out_ref):
    w_aug = jnp.concatenate(
        [
            wt_ref[...],
            b_ref[...],
            jnp.zeros((POOL_W - EMBED_DIM - 1, BV), jnp.float32),
        ],
        axis=0,
    )
    out_ref[...] = lax.dot_general(
        w_aug,
        pooled_ref[...],
        (((0,), (1,)), ((), ())),
        preferred_element_type=jnp.float32,
    )


def _proj_tc(pooled_aug, Wt, b2):
    return pl.pallas_call(
        _proj_body,
        grid=(NB,),
        in_specs=[
            pl.BlockSpec((BATCH, POOL_W), lambda j: (0, 0)),
            pl.BlockSpec((EMBED_DIM, BV), lambda j: (0, j)),
            pl.BlockSpec((1, BV), lambda j: (0, j)),
        ],
        out_specs=pl.BlockSpec((BV, BATCH), lambda j: (j, 0)),
        out_shape=jax.ShapeDtypeStruct((VOCAB, BATCH), jnp.float32),
        compiler_params=pltpu.CompilerParams(
            dimension_semantics=("arbitrary",),
        ),
    )(pooled_aug, Wt, b2)


def kernel(context, emb_table, W, b):
    pooled_aug = _pooled_sc(context, emb_table)
    return _proj_tc(pooled_aug, W.T, b.reshape(1, VOCAB)).T


# speedup vs baseline: 1.4396x; 1.0003x over previous
import functools

import jax
import jax.numpy as jnp
from jax import lax
from jax.experimental import pallas as pl
from jax.experimental.pallas import tpu as pltpu
from jax.experimental.pallas import tpu_sc as plsc

VOCAB = 100000
EMBED_DIM = 64
BATCH = 1024
CTX_LEN = 20

LANES = 16
POOL_W = 128


def _pooled_sc(context, emb_table):
    info = plsc.get_sparse_core_info()
    nw = info.num_cores * info.num_subcores
    b_per_w = BATCH // nw
    d_groups = EMBED_DIM // LANES

    mesh = plsc.VectorSubcoreMesh(core_axis_name="c", subcore_axis_name="s")

    idx_per_w = b_per_w * CTX_LEN
    n_chunks = idx_per_w // 128
    ctx3 = context.reshape(nw, n_chunks, 128)

    @functools.partial(
        pl.kernel,
        out_type=jax.ShapeDtypeStruct((BATCH, POOL_W), jnp.float32),
        mesh=mesh,
        scratch_types=[
            pltpu.VMEM((n_chunks, 128), jnp.int32),
            pltpu.VMEM((idx_per_w, EMBED_DIM), jnp.float32),
            pltpu.VMEM((b_per_w, POOL_W), jnp.float32),
            pltpu.SemaphoreType.DMA,
        ],
        compiler_params=pltpu.CompilerParams(use_tc_tiling_on_sc=False),
    )
    def sc_pool(ctx_hbm, table_hbm, out_hbm, idx_v, rows_v, out_v, sem):
        wid = lax.axis_index("s") * info.num_cores + lax.axis_index("c")
        base = wid * b_per_w
        pltpu.sync_copy(ctx_hbm.at[wid], idx_v)
        copies = []
        for j in range(n_chunks):
            copies.append(
                pltpu.make_async_copy(
                    table_hbm.at[idx_v.at[j]],
                    rows_v.at[pl.ds(j * 128, 128)],
                    sem,
                )
            )
            copies[-1].start()
        for c in copies:
            c.wait()

        scale = jnp.float32(1.0 / CTX_LEN)
        ones_col = jnp.where(
            lax.iota(jnp.int32, LANES) == 0, jnp.float32(1.0), jnp.float32(0.0)
        )

        def body(b, carry):
            rbase = b * CTX_LEN
            for d in range(d_groups):
                sl = pl.ds(d * LANES, LANES)
                acc = rows_v[rbase, sl]
                for l in range(1, CTX_LEN):
                    acc = acc + rows_v[rbase + l, sl]
                out_v[b, sl] = acc * scale
            out_v[b, pl.ds(EMBED_DIM, LANES)] = ones_col
            out_v[b, pl.ds(EMBED_DIM + LANES, LANES)] = ones_col * 0.0
            out_v[b, pl.ds(EMBED_DIM + 2 * LANES, LANES)] = ones_col * 0.0
            out_v[b, pl.ds(EMBED_DIM + 3 * LANES, LANES)] = ones_col * 0.0
            return carry

        lax.fori_loop(0, b_per_w, body, 0)
        pltpu.sync_copy(out_v, out_hbm.at[pl.ds(base, b_per_w)])

    return sc_pool(ctx3, emb_table)


BV = 5120
NB = pl.cdiv(VOCAB, BV)


def _proj_body(pooled_ref, wt_ref, b_ref, out_ref):
    w_aug = jnp.concatenate(
        [
            wt_ref[...],
            b_ref[...],
            jnp.zeros((POOL_W - EMBED_DIM - 1, BV), jnp.float32),
        ],
        axis=0,
    )
    out_ref[...] = lax.dot_general(
        w_aug,
        pooled_ref[...],
        (((0,), (1,)), ((), ())),
        preferred_element_type=jnp.float32,
    )


def _proj_tc(pooled_aug, Wt, b2):
    return pl.pallas_call(
        _proj_body,
        grid=(NB,),
        in_specs=[
            pl.BlockSpec((BATCH, POOL_W), lambda j: (0, 0)),
            pl.BlockSpec((EMBED_DIM, BV), lambda j: (0, j)),
            pl.BlockSpec((1, BV), lambda j: (0, j)),
        ],
        out_specs=pl.BlockSpec((BV, BATCH), lambda j: (j, 0)),
        out_shape=jax.ShapeDtypeStruct((VOCAB, BATCH), jnp.float32),
        compiler_params=pltpu.CompilerParams(
            dimension_semantics=("arbitrary",),
        ),
    )(pooled_aug, Wt, b2)


def kernel(context, emb_table, W, b):
    pooled_aug = _pooled_sc(context, emb_table)
    return _proj_tc(pooled_aug, W.T, b.reshape(1, VOCAB)).T
